# Initial kernel scaffold; baseline (speedup 1.0000x reference)
import jax
import jax.numpy as jnp
from jax.experimental import pallas as pl


def kernel(x, token_table, pos_table):
    raise NotImplementedError("write your pallas kernel here")


# speedup vs baseline: 3.6949x; 3.6949x over previous
import functools

import jax
import jax.numpy as jnp
from jax import lax
from jax.experimental import pallas as pl
from jax.experimental.pallas import tpu as pltpu
from jax.experimental.pallas import tpu_sc as plsc

_LANES = 16
_NC = 2
_NS = 16
_NW = _NC * _NS

_G = 4
_CH = 100


def _emb_body(x_hbm, tok_hbm, pos_hbm, out_hbm, idx_v, rows_v, pos_v, sem,
              *, L, D, rows_per_w):
    wid = lax.axis_index("s") * _NC + lax.axis_index("c")
    nch = (_G * L) // _CH
    groups = rows_per_w // _G
    dchunks = D // _LANES

    pltpu.sync_copy(pos_hbm, pos_v)

    def step(gi, carry):
        idx_row0 = pl.multiple_of((wid * rows_per_w + gi * _G) * L // _CH, 8)
        pltpu.sync_copy(x_hbm.at[pl.ds(idx_row0, nch)], idx_v)

        handles = []
        for j in range(nch):
            handles.append(
                pltpu.async_copy(
                    tok_hbm.at[idx_v.at[j]],
                    rows_v.at[pl.ds(j * _CH, _CH)],
                    sem,
                )
            )
        for h in handles:
            h.wait()

        def add_l(l, c):
            for ci in range(dchunks):
                sl = pl.ds(ci * _LANES, _LANES)
                pc = pos_v[l, sl]
                for g in range(_G):
                    r = g * L + l
                    rows_v[r, sl] = rows_v[r, sl] + pc
            return c

        lax.fori_loop(0, L, add_l, 0)

        out_row0 = (wid * rows_per_w + gi * _G) * L
        pltpu.sync_copy(rows_v, out_hbm.at[pl.ds(out_row0, _G * L)])
        return carry

    lax.fori_loop(0, groups, step, 0)


def kernel(x, token_table, pos_table):
    B, L = x.shape
    V, D = token_table.shape
    rows_per_w = B // _NW
    nch = (_G * L) // _CH

    x_view = x.reshape(B * L // _CH, _CH).astype(jnp.int32)

    body = functools.partial(_emb_body, L=L, D=D, rows_per_w=rows_per_w)
    emb = pl.kernel(
        body,
        out_type=jax.ShapeDtypeStruct((B * L, D), jnp.float32),
        mesh=plsc.VectorSubcoreMesh(core_axis_name="c", subcore_axis_name="s"),
        scratch_types=[
            pltpu.VMEM((nch, _CH), jnp.int32),
            pltpu.VMEM((_G * L, D), jnp.float32),
            pltpu.VMEM((L, D), jnp.float32),
            pltpu.SemaphoreType.DMA,
        ],
        compiler_params=pltpu.CompilerParams(use_tc_tiling_on_sc=False),
    )
    out = emb(x_view, token_table, pos_table)
    return out.reshape(B, L, D)


# speedup vs baseline: 4.1081x; 1.1118x over previous
import functools

import jax
import jax.numpy as jnp
from jax import lax
from jax.experimental import pallas as pl
from jax.experimental.pallas import tpu as pltpu
from jax.experimental.pallas import tpu_sc as plsc

_LANES = 16
_NC = 2
_NS = 16
_NW = _NC * _NS

_G = 4
_CH = 100


def _emb_body(x_hbm, tok_hbm, pos_hbm, out_hbm,
              idx0, idx1, rows0, rows1, pos_v,
              gsem0, gsem1, osem0, osem1,
              *, L, D, rows_per_w):
    wid = lax.axis_index("s") * _NC + lax.axis_index("c")
    nch = (_G * L) // _CH
    groups = rows_per_w // _G
    dchunks = D // _LANES
    row0_w = wid * rows_per_w * L

    def fire(gi, idx_v, rows_v, gsem):
        idx_row0 = pl.multiple_of((row0_w + gi * _G * L) // _CH, 8)
        pltpu.sync_copy(x_hbm.at[pl.ds(idx_row0, nch)], idx_v)
        for j in range(nch):
            pltpu.async_copy(
                tok_hbm.at[idx_v.at[j]],
                rows_v.at[pl.ds(j * _CH, _CH)],
                gsem,
            )

    def wait_gathers(rows_v, gsem):
        pltpu.make_async_copy(tok_hbm.at[pl.ds(0, _G * L)], rows_v, gsem).wait()

    def fire_out(gi, rows_v, osem):
        pltpu.async_copy(rows_v, out_hbm.at[pl.ds(row0_w + gi * _G * L, _G * L)],
                         osem)

    def wait_out(rows_v, osem):
        pltpu.make_async_copy(rows_v, out_hbm.at[pl.ds(0, _G * L)], osem).wait()

    def add_pos(rows_v):
        def add_l(l, c):
            for ci in range(dchunks):
                sl = pl.ds(ci * _LANES, _LANES)
                pc = pos_v[l, sl]
                for g in range(_G):
                    r = g * L + l
                    rows_v[r, sl] = rows_v[r, sl] + pc
            return c
        lax.fori_loop(0, L, add_l, 0)

    pltpu.sync_copy(pos_hbm, pos_v)
    fire(0, idx0, rows0, gsem0)

    def pair(p, carry):
        gi0 = 2 * p
        gi1 = 2 * p + 1
        @pl.when(p > 0)
        def _():
            wait_out(rows1, osem1)
        fire(gi1, idx1, rows1, gsem1)
        wait_gathers(rows0, gsem0)
        add_pos(rows0)
        fire_out(gi0, rows0, osem0)
        wait_out(rows0, osem0)
        @pl.when(p < groups // 2 - 1)
        def _():
            fire(gi1 + 1, idx0, rows0, gsem0)
        wait_gathers(rows1, gsem1)
        add_pos(rows1)
        fire_out(gi1, rows1, osem1)
        return carry

    lax.fori_loop(0, groups // 2, pair, 0)
    wait_out(rows1, osem1)


def kernel(x, token_table, pos_table):
    B, L = x.shape
    V, D = token_table.shape
    rows_per_w = B // _NW
    nch = (_G * L) // _CH

    x_view = x.reshape(B * L // _CH, _CH).astype(jnp.int32)

    body = functools.partial(_emb_body, L=L, D=D, rows_per_w=rows_per_w)
    emb = pl.kernel(
        body,
        out_type=jax.ShapeDtypeStruct((B * L, D), jnp.float32),
        mesh=plsc.VectorSubcoreMesh(core_axis_name="c", subcore_axis_name="s"),
        scratch_types=[
            pltpu.VMEM((nch, _CH), jnp.int32),
            pltpu.VMEM((nch, _CH), jnp.int32),
            pltpu.VMEM((_G * L, D), jnp.float32),
            pltpu.VMEM((_G * L, D), jnp.float32),
            pltpu.VMEM((L, D), jnp.float32),
            pltpu.SemaphoreType.DMA,
            pltpu.SemaphoreType.DMA,
            pltpu.SemaphoreType.DMA,
            pltpu.SemaphoreType.DMA,
        ],
        compiler_params=pltpu.CompilerParams(use_tc_tiling_on_sc=False),
    )
    out = emb(x_view, token_table, pos_table)
    return out.reshape(B, L, D)


# speedup vs baseline: 4.1111x; 1.0007x over previous
import functools

import jax
import jax.numpy as jnp
from jax import lax
from jax.experimental import pallas as pl
from jax.experimental.pallas import tpu as pltpu
from jax.experimental.pallas import tpu_sc as plsc

_LANES = 16
_NC = 2
_NS = 16
_NW = _NC * _NS

_G = 4
_CH = 100


def _emb_body(x_hbm, tok_hbm, pos_hbm, out_hbm,
              idx0, idx1, rows0, rows1, pos_v,
              gsem0, gsem1, osem0, osem1,
              *, L, D, rows_per_w):
    wid = lax.axis_index("s") * _NC + lax.axis_index("c")
    nch = (_G * L) // _CH
    groups = rows_per_w // _G
    dchunks = D // _LANES
    b0_w = wid * rows_per_w

    def fire(gi, idx_v, rows_v, gsem):
        idx_row0 = pl.multiple_of((b0_w + gi * _G) * L // _CH, 8)
        pltpu.sync_copy(x_hbm.at[pl.ds(idx_row0, nch)], idx_v)
        for j in range(nch):
            pltpu.async_copy(
                tok_hbm.at[idx_v.at[j]],
                rows_v.at[j // 2].at[pl.ds((j % 2) * _CH, _CH)],
                gsem,
            )

    def wait_gathers(rows_v, gsem):
        pltpu.make_async_copy(out_hbm.at[pl.ds(0, _G)], rows_v, gsem).wait()

    def fire_out(gi, rows_v, osem):
        pltpu.async_copy(rows_v, out_hbm.at[pl.ds(b0_w + gi * _G, _G)], osem)

    def wait_out(rows_v, osem):
        pltpu.make_async_copy(rows_v, out_hbm.at[pl.ds(0, _G)], osem).wait()

    def add_pos(rows_v):
        def add_l(l, c):
            for ci in range(dchunks):
                sl = pl.ds(ci * _LANES, _LANES)
                pc = pos_v[l, sl]
                for g in range(_G):
                    rows_v[g, l, sl] = rows_v[g, l, sl] + pc
            return c
        lax.fori_loop(0, L, add_l, 0)

    pltpu.sync_copy(pos_hbm, pos_v)
    fire(0, idx0, rows0, gsem0)

    def pair(p, carry):
        gi0 = 2 * p
        gi1 = 2 * p + 1
        @pl.when(p > 0)
        def _():
            wait_out(rows1, osem1)
        fire(gi1, idx1, rows1, gsem1)
        wait_gathers(rows0, gsem0)
        add_pos(rows0)
        fire_out(gi0, rows0, osem0)
        wait_out(rows0, osem0)
        @pl.when(p < groups // 2 - 1)
        def _():
            fire(gi1 + 1, idx0, rows0, gsem0)
        wait_gathers(rows1, gsem1)
        add_pos(rows1)
        fire_out(gi1, rows1, osem1)
        return carry

    lax.fori_loop(0, groups // 2, pair, 0)
    wait_out(rows1, osem1)


def kernel(x, token_table, pos_table):
    B, L = x.shape
    V, D = token_table.shape
    rows_per_w = B // _NW
    nch = (_G * L) // _CH

    x_view = x.reshape(B * L // _CH, _CH).astype(jnp.int32)

    body = functools.partial(_emb_body, L=L, D=D, rows_per_w=rows_per_w)
    emb = pl.kernel(
        body,
        out_type=jax.ShapeDtypeStruct((B, L, D), jnp.float32),
        mesh=plsc.VectorSubcoreMesh(core_axis_name="c", subcore_axis_name="s"),
        scratch_types=[
            pltpu.VMEM((nch, _CH), jnp.int32),
            pltpu.VMEM((nch, _CH), jnp.int32),
            pltpu.VMEM((_G, L, D), jnp.float32),
            pltpu.VMEM((_G, L, D), jnp.float32),
            pltpu.VMEM((L, D), jnp.float32),
            pltpu.SemaphoreType.DMA,
            pltpu.SemaphoreType.DMA,
            pltpu.SemaphoreType.DMA,
            pltpu.SemaphoreType.DMA,
        ],
        compiler_params=pltpu.CompilerParams(use_tc_tiling_on_sc=False),
    )
    return emb(x_view, token_table, pos_table)
